# baseline (speedup 1.0000x reference)
import functools

import jax
import jax.numpy as jnp
from jax import lax
from jax.experimental import pallas as pl
from jax.experimental.pallas import tpu as pltpu
from jax.experimental.pallas import tpu_sc as plsc

NC = 2
NS = 16
LANES = 16
CHUNK = 128


def _sc_mesh():
    return plsc.VectorSubcoreMesh(core_axis_name="c", subcore_axis_name="s")


def _deg_hist(cpt, NPAD):
    seg = NPAD // NS
    K = 8

    @functools.partial(
        pl.kernel,
        out_type=jax.ShapeDtypeStruct((NC * NPAD,), jnp.float32),
        mesh=_sc_mesh(),
        scratch_types=[
            pltpu.VMEM((cpt, CHUNK), jnp.int32),
            pltpu.VMEM((CHUNK,), jnp.float32),
            pltpu.VMEM((seg,), jnp.float32),
            pltpu.VMEM_SHARED((NPAD,), jnp.float32),
            pltpu.SemaphoreType.DMA,
        ],
    )
    def k(row2d_hbm, out_hbm, idx_v, ones_v, z_v, deg_sh, sem):
        c = lax.axis_index("c")
        s = lax.axis_index("s")
        wid = s * NC + c

        for j in range(CHUNK // LANES):
            ones_v[pl.ds(j * LANES, LANES)] = jnp.ones((LANES,), jnp.float32)

        @pl.loop(0, seg // LANES)
        def _(j):
            z_v[pl.ds(j * LANES, LANES)] = jnp.zeros((LANES,), jnp.float32)

        hidx = pltpu.async_copy(row2d_hbm.at[pl.ds(wid * cpt, cpt)], idx_v, sem)
        pltpu.sync_copy(z_v, deg_sh.at[pl.ds(s * seg, seg)])
        hidx.wait()
        plsc.subcore_barrier()

        @pl.loop(0, cpt // K)
        def _(g):
            hs = [
                pltpu.async_copy(
                    ones_v, deg_sh.at[idx_v.at[g * K + t]], sem, add=True
                )
                for t in range(K)
            ]
            for h in hs:
                h.wait()

        plsc.subcore_barrier()
        pltpu.sync_copy(deg_sh.at[pl.ds(s * seg, seg)], z_v)
        pltpu.sync_copy(z_v, out_hbm.at[pl.ds(c * NPAD + s * seg, seg)])

    return k


def _edge_scatter(cpt, NPAD, D):
    rps = NPAD // NS
    hp = cpt // 2

    @functools.partial(
        pl.kernel,
        out_type=jax.ShapeDtypeStruct((NC, NPAD, D), jnp.float32),
        mesh=_sc_mesh(),
        scratch_types=[
            pltpu.VMEM((hp, CHUNK), jnp.int32),
            pltpu.VMEM((hp, CHUNK), jnp.int32),
            pltpu.VMEM((CHUNK, D), jnp.float32),
            pltpu.VMEM((CHUNK, D), jnp.float32),
            pltpu.VMEM_SHARED((NPAD, D), jnp.float32),
            pltpu.SemaphoreType.DMA,
            pltpu.SemaphoreType.DMA,
        ],
    )
    def k(row2d, col2d, y_hbm, out_hbm, ridx, cidx, rows0, rows1, acc_sh,
          gsem, isem):
        c = lax.axis_index("c")
        s = lax.axis_index("s")
        wid = s * NC + c
        nfull, remr = rps // CHUNK, rps % CHUNK

        def gather(idx_ref, buf):
            return pltpu.async_copy(y_hbm.at[idx_ref], buf, gsem)

        def wait_gather(buf):
            pltpu.make_async_copy(y_hbm.at[ridx.at[0]], buf, gsem).wait()

        def scatter(buf, idx_ref):
            pltpu.sync_copy(buf, acc_sh.at[idx_ref], add=True)

        @pl.loop(0, CHUNK)
        def _(i):
            for j in range(D // LANES):
                rows0[i, pl.ds(j * LANES, LANES)] = jnp.zeros(
                    (LANES,), jnp.float32
                )

        for t in range(nfull):
            pltpu.sync_copy(rows0, acc_sh.at[pl.ds(s * rps + t * CHUNK, CHUNK)])
        if remr:
            pltpu.sync_copy(
                rows0.at[pl.ds(0, remr)],
                acc_sh.at[pl.ds(s * rps + nfull * CHUNK, remr)],
            )
        plsc.subcore_barrier()

        for ph in range(2):
            h1 = pltpu.async_copy(
                row2d.at[pl.ds(wid * cpt + ph * hp, hp)], ridx, isem
            )
            h2 = pltpu.async_copy(
                col2d.at[pl.ds(wid * cpt + ph * hp, hp)], cidx, isem
            )
            h1.wait()
            h2.wait()
            gather(ridx.at[0], rows0)

            @pl.loop(0, hp // 2 - 1)
            def _(p):
                j0 = 2 * p
                wait_gather(rows0)
                gather(ridx.at[j0 + 1], rows1)
                scatter(rows0, cidx.at[j0])
                gather(ridx.at[j0 + 2], rows0)
                wait_gather(rows1)
                scatter(rows1, cidx.at[j0 + 1])

            wait_gather(rows0)
            gather(ridx.at[hp - 1], rows1)
            scatter(rows0, cidx.at[hp - 2])
            wait_gather(rows1)
            scatter(rows1, cidx.at[hp - 1])

        plsc.subcore_barrier()
        for t in range(nfull):
            sl = pl.ds(s * rps + t * CHUNK, CHUNK)
            pltpu.sync_copy(acc_sh.at[sl], rows0)
            pltpu.sync_copy(rows0, out_hbm.at[c, sl])
        if remr:
            sl = pl.ds(s * rps + nfull * CHUNK, remr)
            pltpu.sync_copy(acc_sh.at[sl], rows0.at[pl.ds(0, remr)])
            pltpu.sync_copy(rows0.at[pl.ds(0, remr)], out_hbm.at[c, sl])

    return k


def _dis_from_deg(deg_blk):
    deg = deg_blk[:, 0:1] + deg_blk[:, 1:2]
    return jnp.where(deg > 0, lax.rsqrt(deg), 0.0)


def _y_body(deg_ref, x_ref, y_ref):
    y_ref[...] = jnp.maximum(x_ref[...], 0.0) * _dis_from_deg(deg_ref[...])


def _out_body(aw_ref, deg_ref, x_ref, acc_ref, o_ref):
    dis = _dis_from_deg(deg_ref[...])
    o_ref[...] = x_ref[...] + aw_ref[0] * dis * (acc_ref[0] + acc_ref[1])


def kernel(x, edge_index, num_nodes, adaptive_weight):
    N, D = x.shape
    E = edge_index.shape[1]
    row = edge_index[0]
    col = edge_index[1]
    aw = jnp.reshape(adaptive_weight, (1,)).astype(jnp.float32)

    npad = -(-N // (NS * 8)) * (NS * 8)
    dump = npad - 1
    nw = NC * NS

    cpt = -(-E // (nw * CHUNK))
    cpt += cpt % 2
    epad = nw * cpt * CHUNK
    row_g = jnp.concatenate([row, jnp.zeros((epad - E,), jnp.int32)])
    row_h = jnp.concatenate([row, jnp.full((epad - E,), dump, jnp.int32)])
    col_p = jnp.concatenate([col, jnp.full((epad - E,), dump, jnp.int32)])
    row_g2 = row_g.reshape(-1, CHUNK)
    row_h2 = row_h.reshape(-1, CHUNK)
    col_p2 = col_p.reshape(-1, CHUNK)

    deg2 = _deg_hist(cpt, npad)(row_h2).reshape(NC, npad)
    deg_t = deg2.T[:N]

    R = 400
    grid = N // R
    y = pl.pallas_call(
        _y_body,
        grid=(grid,),
        in_specs=[
            pl.BlockSpec((R, NC), lambda i: (i, 0)),
            pl.BlockSpec((R, D), lambda i: (i, 0)),
        ],
        out_specs=pl.BlockSpec((R, D), lambda i: (i, 0)),
        out_shape=jax.ShapeDtypeStruct((N, D), jnp.float32),
    )(deg_t, x)

    acc2 = _edge_scatter(cpt, npad, D)(row_g2, col_p2, y)

    out = pl.pallas_call(
        _out_body,
        grid=(grid,),
        in_specs=[
            pl.BlockSpec(memory_space=pltpu.SMEM),
            pl.BlockSpec((R, NC), lambda i: (i, 0)),
            pl.BlockSpec((R, D), lambda i: (i, 0)),
            pl.BlockSpec((NC, R, D), lambda i: (0, i, 0)),
        ],
        out_specs=pl.BlockSpec((R, D), lambda i: (i, 0)),
        out_shape=jax.ShapeDtypeStruct((N, D), jnp.float32),
    )(aw, deg_t, x, acc2)
    return out


# speedup vs baseline: 1.0126x; 1.0126x over previous
import functools

import jax
import jax.numpy as jnp
from jax import lax
from jax.experimental import pallas as pl
from jax.experimental.pallas import tpu as pltpu
from jax.experimental.pallas import tpu_sc as plsc

NC = 2
NS = 16
LANES = 16
CHUNK = 128


def _sc_mesh():
    return plsc.VectorSubcoreMesh(core_axis_name="c", subcore_axis_name="s")


def _deg_hist(cpt, NPAD):
    seg = NPAD // NS
    K = 8

    @functools.partial(
        pl.kernel,
        out_type=jax.ShapeDtypeStruct((NC * NPAD,), jnp.float32),
        mesh=_sc_mesh(),
        scratch_types=[
            pltpu.VMEM((cpt, CHUNK), jnp.int32),
            pltpu.VMEM((CHUNK,), jnp.float32),
            pltpu.VMEM((seg,), jnp.float32),
            pltpu.VMEM_SHARED((NPAD,), jnp.float32),
            pltpu.SemaphoreType.DMA,
        ],
    )
    def k(row2d_hbm, out_hbm, idx_v, ones_v, z_v, deg_sh, sem):
        c = lax.axis_index("c")
        s = lax.axis_index("s")
        wid = s * NC + c

        for j in range(CHUNK // LANES):
            ones_v[pl.ds(j * LANES, LANES)] = jnp.ones((LANES,), jnp.float32)

        @pl.loop(0, seg // LANES)
        def _(j):
            z_v[pl.ds(j * LANES, LANES)] = jnp.zeros((LANES,), jnp.float32)

        hidx = pltpu.async_copy(row2d_hbm.at[pl.ds(wid * cpt, cpt)], idx_v, sem)
        pltpu.sync_copy(z_v, deg_sh.at[pl.ds(s * seg, seg)])
        hidx.wait()
        plsc.subcore_barrier()

        @pl.loop(0, cpt // K)
        def _(g):
            hs = [
                pltpu.async_copy(
                    ones_v, deg_sh.at[idx_v.at[g * K + t]], sem, add=True
                )
                for t in range(K)
            ]
            for h in hs:
                h.wait()

        plsc.subcore_barrier()
        pltpu.sync_copy(deg_sh.at[pl.ds(s * seg, seg)], z_v)
        pltpu.sync_copy(z_v, out_hbm.at[pl.ds(c * NPAD + s * seg, seg)])

    return k


def _edge_scatter(cpt, NPAD, D):
    rps = NPAD // NS
    hp = cpt // 2

    @functools.partial(
        pl.kernel,
        out_type=jax.ShapeDtypeStruct((NC, NPAD, D), jnp.float32),
        mesh=_sc_mesh(),
        scratch_types=[
            pltpu.VMEM((hp, CHUNK), jnp.int32),
            pltpu.VMEM((hp, CHUNK), jnp.int32),
            pltpu.VMEM((CHUNK, D), jnp.float32),
            pltpu.VMEM((CHUNK, D), jnp.float32),
            pltpu.VMEM_SHARED((NPAD, D), jnp.float32),
            pltpu.SemaphoreType.DMA,
            pltpu.SemaphoreType.DMA,
        ],
    )
    def k(row2d, col2d, y_hbm, out_hbm, ridx, cidx, rows0, rows1, acc_sh,
          gsem, isem):
        c = lax.axis_index("c")
        s = lax.axis_index("s")
        wid = s * NC + c
        nfull, remr = rps // CHUNK, rps % CHUNK

        def gather(idx_ref, buf):
            return pltpu.async_copy(y_hbm.at[idx_ref], buf, gsem)

        def wait_gather(buf):
            pltpu.make_async_copy(y_hbm.at[ridx.at[0]], buf, gsem).wait()

        def scatter(buf, idx_ref):
            pltpu.sync_copy(buf, acc_sh.at[idx_ref], add=True)

        @pl.loop(0, CHUNK)
        def _(i):
            for j in range(D // LANES):
                rows0[i, pl.ds(j * LANES, LANES)] = jnp.zeros(
                    (LANES,), jnp.float32
                )

        for t in range(nfull):
            pltpu.sync_copy(rows0, acc_sh.at[pl.ds(s * rps + t * CHUNK, CHUNK)])
        if remr:
            pltpu.sync_copy(
                rows0.at[pl.ds(0, remr)],
                acc_sh.at[pl.ds(s * rps + nfull * CHUNK, remr)],
            )
        plsc.subcore_barrier()

        for ph in range(2):
            h1 = pltpu.async_copy(
                row2d.at[pl.ds(wid * cpt + ph * hp, hp)], ridx, isem
            )
            h2 = pltpu.async_copy(
                col2d.at[pl.ds(wid * cpt + ph * hp, hp)], cidx, isem
            )
            h1.wait()
            h2.wait()
            gather(ridx.at[0], rows0)

            @pl.loop(0, hp // 2 - 1)
            def _(p):
                j0 = 2 * p
                wait_gather(rows0)
                gather(ridx.at[j0 + 1], rows1)
                scatter(rows0, cidx.at[j0])
                gather(ridx.at[j0 + 2], rows0)
                wait_gather(rows1)
                scatter(rows1, cidx.at[j0 + 1])

            wait_gather(rows0)
            gather(ridx.at[hp - 1], rows1)
            scatter(rows0, cidx.at[hp - 2])
            wait_gather(rows1)
            scatter(rows1, cidx.at[hp - 1])

        plsc.subcore_barrier()
        for t in range(nfull):
            sl = pl.ds(s * rps + t * CHUNK, CHUNK)
            pltpu.sync_copy(acc_sh.at[sl], rows0)
            pltpu.sync_copy(rows0, out_hbm.at[c, sl])
        if remr:
            sl = pl.ds(s * rps + nfull * CHUNK, remr)
            pltpu.sync_copy(acc_sh.at[sl], rows0.at[pl.ds(0, remr)])
            pltpu.sync_copy(rows0.at[pl.ds(0, remr)], out_hbm.at[c, sl])

    return k


def _dis_from_deg(deg_blk):
    deg = deg_blk[:, 0:1] + deg_blk[:, 1:2]
    return jnp.where(deg > 0, lax.rsqrt(deg), 0.0)


def _y_body(deg_ref, x_ref, y_ref):
    y_ref[...] = jnp.maximum(x_ref[...], 0.0) * _dis_from_deg(deg_ref[...])


def _out_body(aw_ref, deg_ref, x_ref, acc_ref, o_ref):
    dis = _dis_from_deg(deg_ref[...])
    o_ref[...] = x_ref[...] + aw_ref[0] * dis * (acc_ref[0] + acc_ref[1])


def kernel(x, edge_index, num_nodes, adaptive_weight):
    N, D = x.shape
    E = edge_index.shape[1]
    row = edge_index[0]
    col = edge_index[1]
    aw = jnp.reshape(adaptive_weight, (1,)).astype(jnp.float32)

    npad = -(-N // (NS * 8)) * (NS * 8)
    nw = NC * NS

    cpt = -(-E // (nw * CHUNK))
    cpt += cpt % 2
    epad = nw * cpt * CHUNK
    dump_idx = N + jnp.arange(epad - E, dtype=jnp.int32) % (npad - N)
    row_g = jnp.concatenate([row, jnp.zeros((epad - E,), jnp.int32)])
    row_h = jnp.concatenate([row, dump_idx])
    col_p = jnp.concatenate([col, dump_idx])
    row_g2 = row_g.reshape(-1, CHUNK)
    row_h2 = row_h.reshape(-1, CHUNK)
    col_p2 = col_p.reshape(-1, CHUNK)

    deg2 = _deg_hist(cpt, npad)(row_h2).reshape(NC, npad)
    deg_t = deg2.T[:N]

    R = 400
    grid = N // R
    y = pl.pallas_call(
        _y_body,
        grid=(grid,),
        in_specs=[
            pl.BlockSpec((R, NC), lambda i: (i, 0)),
            pl.BlockSpec((R, D), lambda i: (i, 0)),
        ],
        out_specs=pl.BlockSpec((R, D), lambda i: (i, 0)),
        out_shape=jax.ShapeDtypeStruct((N, D), jnp.float32),
    )(deg_t, x)

    acc2 = _edge_scatter(cpt, npad, D)(row_g2, col_p2, y)

    out = pl.pallas_call(
        _out_body,
        grid=(grid,),
        in_specs=[
            pl.BlockSpec(memory_space=pltpu.SMEM),
            pl.BlockSpec((R, NC), lambda i: (i, 0)),
            pl.BlockSpec((R, D), lambda i: (i, 0)),
            pl.BlockSpec((NC, R, D), lambda i: (0, i, 0)),
        ],
        out_specs=pl.BlockSpec((R, D), lambda i: (i, 0)),
        out_shape=jax.ShapeDtypeStruct((N, D), jnp.float32),
    )(aw, deg_t, x, acc2)
    return out


# speedup vs baseline: 2.8651x; 2.8295x over previous
import functools

import jax
import jax.numpy as jnp
from jax import lax
from jax.experimental import pallas as pl
from jax.experimental.pallas import tpu as pltpu
from jax.experimental.pallas import tpu_sc as plsc

NC = 2
NS = 16
LANES = 16
CHUNK = 128


def _sc_mesh():
    return plsc.VectorSubcoreMesh(core_axis_name="c", subcore_axis_name="s")


def _deg_hist(cpt, NPAD):
    seg = NPAD // NS
    K = 8

    @functools.partial(
        pl.kernel,
        out_type=jax.ShapeDtypeStruct((NC * NPAD,), jnp.float32),
        mesh=_sc_mesh(),
        scratch_types=[
            pltpu.VMEM((cpt, CHUNK), jnp.int32),
            pltpu.VMEM((CHUNK,), jnp.float32),
            pltpu.VMEM((seg,), jnp.float32),
            pltpu.VMEM_SHARED((NPAD,), jnp.float32),
            pltpu.SemaphoreType.DMA,
        ],
    )
    def k(row2d_hbm, out_hbm, idx_v, ones_v, z_v, deg_sh, sem):
        c = lax.axis_index("c")
        s = lax.axis_index("s")
        wid = s * NC + c

        for j in range(CHUNK // LANES):
            ones_v[pl.ds(j * LANES, LANES)] = jnp.ones((LANES,), jnp.float32)

        @pl.loop(0, seg // LANES)
        def _(j):
            z_v[pl.ds(j * LANES, LANES)] = jnp.zeros((LANES,), jnp.float32)

        hidx = pltpu.async_copy(row2d_hbm.at[pl.ds(wid * cpt, cpt)], idx_v, sem)
        pltpu.sync_copy(z_v, deg_sh.at[pl.ds(s * seg, seg)])
        hidx.wait()
        plsc.subcore_barrier()

        @pl.loop(0, cpt // K)
        def _(g):
            hs = [
                pltpu.async_copy(
                    ones_v, deg_sh.at[idx_v.at[g * K + t]], sem, add=True
                )
                for t in range(K)
            ]
            for h in hs:
                h.wait()

        plsc.subcore_barrier()
        pltpu.sync_copy(deg_sh.at[pl.ds(s * seg, seg)], z_v)
        pltpu.sync_copy(z_v, out_hbm.at[pl.ds(c * NPAD + s * seg, seg)])

    return k


def _edge_scatter(cpt, NPAD, D):
    rps = NPAD // NS
    hp = cpt // 2

    @functools.partial(
        pl.kernel,
        out_type=jax.ShapeDtypeStruct((NC, NPAD, D), jnp.float32),
        mesh=_sc_mesh(),
        scratch_types=[
            pltpu.VMEM((hp, CHUNK), jnp.int32),
            pltpu.VMEM((hp, CHUNK), jnp.int32),
            pltpu.VMEM((CHUNK, D), jnp.float32),
            pltpu.VMEM((CHUNK, D), jnp.float32),
            pltpu.VMEM_SHARED((NPAD, D), jnp.float32),
            pltpu.SemaphoreType.DMA,
            pltpu.SemaphoreType.DMA,
        ],
    )
    def k(row2d, col2d, y_hbm, out_hbm, ridx, cidx, rows0, rows1, acc_sh,
          gsem, isem):
        c = lax.axis_index("c")
        s = lax.axis_index("s")
        wid = s * NC + c
        nfull, remr = rps // CHUNK, rps % CHUNK

        def gather(idx_ref, buf):
            return pltpu.async_copy(y_hbm.at[idx_ref], buf, gsem)

        def wait_gather(buf):
            pltpu.make_async_copy(y_hbm.at[ridx.at[0]], buf, gsem).wait()

        def scatter(buf, idx_ref):
            pltpu.sync_copy(buf, acc_sh.at[idx_ref], add=True)

        @pl.loop(0, CHUNK)
        def _(i):
            for j in range(D // LANES):
                rows0[i, pl.ds(j * LANES, LANES)] = jnp.zeros(
                    (LANES,), jnp.float32
                )

        for t in range(nfull):
            pltpu.sync_copy(rows0, acc_sh.at[pl.ds(s * rps + t * CHUNK, CHUNK)])
        if remr:
            pltpu.sync_copy(
                rows0.at[pl.ds(0, remr)],
                acc_sh.at[pl.ds(s * rps + nfull * CHUNK, remr)],
            )
        plsc.subcore_barrier()

        for ph in range(2):
            h1 = pltpu.async_copy(
                row2d.at[pl.ds(wid * cpt + ph * hp, hp)], ridx, isem
            )
            h2 = pltpu.async_copy(
                col2d.at[pl.ds(wid * cpt + ph * hp, hp)], cidx, isem
            )
            h1.wait()
            h2.wait()
            gather(ridx.at[0], rows0)

            @pl.loop(0, hp // 2 - 1)
            def _(p):
                j0 = 2 * p
                wait_gather(rows0)
                gather(ridx.at[j0 + 1], rows1)
                scatter(rows0, cidx.at[j0])
                gather(ridx.at[j0 + 2], rows0)
                wait_gather(rows1)
                scatter(rows1, cidx.at[j0 + 1])

            wait_gather(rows0)
            gather(ridx.at[hp - 1], rows1)
            scatter(rows0, cidx.at[hp - 2])
            wait_gather(rows1)
            scatter(rows1, cidx.at[hp - 1])

        plsc.subcore_barrier()
        for t in range(nfull):
            sl = pl.ds(s * rps + t * CHUNK, CHUNK)
            pltpu.sync_copy(acc_sh.at[sl], rows0)
            pltpu.sync_copy(rows0, out_hbm.at[c, sl])
        if remr:
            sl = pl.ds(s * rps + nfull * CHUNK, remr)
            pltpu.sync_copy(acc_sh.at[sl], rows0.at[pl.ds(0, remr)])
            pltpu.sync_copy(rows0.at[pl.ds(0, remr)], out_hbm.at[c, sl])

    return k


def _dis_from_deg(deg_blk):
    deg = deg_blk[:, 0:1] + deg_blk[:, 1:2]
    return jnp.where(deg > 0, lax.rsqrt(deg), 0.0)


def _y_body(deg_ref, x_ref, y_ref):
    y_ref[...] = jnp.maximum(x_ref[...], 0.0) * _dis_from_deg(deg_ref[...])


def _out_body(aw_ref, deg_ref, x_ref, acc_ref, o_ref):
    dis = _dis_from_deg(deg_ref[...])
    o_ref[...] = x_ref[...] + aw_ref[0] * dis * (acc_ref[0] + acc_ref[1])


def kernel(x, edge_index, num_nodes, adaptive_weight):
    N, D = x.shape
    E = edge_index.shape[1]
    row = edge_index[0]
    col = edge_index[1]
    aw = jnp.reshape(adaptive_weight, (1,)).astype(jnp.float32)

    npad = -(-N // (NS * 8)) * (NS * 8)
    nw = NC * NS

    cpt = -(-E // (nw * CHUNK))
    cpt += cpt % 2
    epad = nw * cpt * CHUNK
    dump_idx = N + jnp.arange(epad - E, dtype=jnp.int32) % (npad - N)
    spread_idx = jnp.arange(epad - E, dtype=jnp.int32) * 79 % N
    row_g = jnp.concatenate([row, spread_idx])
    row_h = jnp.concatenate([row, dump_idx])
    col_p = jnp.concatenate([col, dump_idx])
    row_g2 = row_g.reshape(-1, CHUNK)
    row_h2 = row_h.reshape(-1, CHUNK)
    col_p2 = col_p.reshape(-1, CHUNK)

    deg2 = _deg_hist(cpt, npad)(row_h2).reshape(NC, npad)
    deg_t = deg2.T[:N]

    R = 400
    grid = N // R
    y = pl.pallas_call(
        _y_body,
        grid=(grid,),
        in_specs=[
            pl.BlockSpec((R, NC), lambda i: (i, 0)),
            pl.BlockSpec((R, D), lambda i: (i, 0)),
        ],
        out_specs=pl.BlockSpec((R, D), lambda i: (i, 0)),
        out_shape=jax.ShapeDtypeStruct((N, D), jnp.float32),
    )(deg_t, x)

    acc2 = _edge_scatter(cpt, npad, D)(row_g2, col_p2, y)

    out = pl.pallas_call(
        _out_body,
        grid=(grid,),
        in_specs=[
            pl.BlockSpec(memory_space=pltpu.SMEM),
            pl.BlockSpec((R, NC), lambda i: (i, 0)),
            pl.BlockSpec((R, D), lambda i: (i, 0)),
            pl.BlockSpec((NC, R, D), lambda i: (0, i, 0)),
        ],
        out_specs=pl.BlockSpec((R, D), lambda i: (i, 0)),
        out_shape=jax.ShapeDtypeStruct((N, D), jnp.float32),
    )(aw, deg_t, x, acc2)
    return out
